# Initial kernel scaffold; baseline (speedup 1.0000x reference)
import jax
import jax.numpy as jnp
from jax.experimental import pallas as pl


def kernel(types, x_dopant, node_dopant_index, radii, x_layer_idx, edge_index, batch, embed_table, W_film, b_film, sigma, W_bilin, W_src, b_src, W_dst, b_dst, att, bias_out):
    raise NotImplementedError("write your pallas kernel here")


# speedup vs baseline: 7.6555x; 7.6555x over previous
import functools

import jax
import jax.numpy as jnp
import numpy as np
from jax import lax
from jax.experimental import pallas as pl
from jax.experimental.pallas import tpu as pltpu
from jax.experimental.pallas import tpu_sc as plsc
from jax.scipy.special import erf

_NC = 2
_NS16 = 16
_LANES = 16

_NG = 256


def _k1_body(ndi_hbm, xd_hbm, xli_hbm, radii_hbm, out_hbm,
             xd_v, xli_v, radii_v, idx_v, nfbuf_v):
  cid = lax.axis_index("c")
  sid = lax.axis_index("s")
  wid = sid * _NC + cid

  n = ndi_hbm.shape[0]
  n_groups = n // _LANES
  nw = _NC * _NS16
  base_g = n_groups // nw
  extra = n_groups % nw
  n_g = base_g + jnp.where(wid < extra, 1, 0)

  pltpu.sync_copy(xd_hbm, xd_v)
  pltpu.sync_copy(xli_hbm, xli_v)
  pltpu.sync_copy(radii_hbm, radii_v)

  def group_body(t, carry):
    g = wid + t * nw
    base = g * _LANES
    pltpu.sync_copy(ndi_hbm.at[pl.ds(base, _LANES)], idx_v)
    idx = idx_v[...]
    c0 = plsc.load_gather(xd_v, [idx * 2])
    c1 = plsc.load_gather(xd_v, [idx * 2 + 1])
    li = plsc.load_gather(xli_v, [idx])
    r0 = plsc.load_gather(radii_v, [li * 4])
    r1 = plsc.load_gather(radii_v, [li * 4 + 1])
    r2 = plsc.load_gather(radii_v, [li * 4 + 2])
    r3 = plsc.load_gather(radii_v, [li * 4 + 3])
    nfbuf_v[0, :] = c0
    nfbuf_v[1, :] = c1
    nfbuf_v[2, :] = r0
    nfbuf_v[3, :] = r1
    nfbuf_v[4, :] = r2
    nfbuf_v[5, :] = r3
    z = jnp.zeros((_LANES,), jnp.float32)
    nfbuf_v[6, :] = z
    nfbuf_v[7, :] = z
    pltpu.sync_copy(nfbuf_v, out_hbm.at[:, pl.ds(base, _LANES)])
    return carry

  lax.fori_loop(0, n_g, group_body, 0)


def _k1_call(ndi, xd_flat, xli, radii_flat):
  n = ndi.shape[0]
  mesh = plsc.VectorSubcoreMesh(core_axis_name="c", subcore_axis_name="s",
                                num_cores=_NC, num_subcores=_NS16)
  return pl.kernel(
      _k1_body,
      out_type=jax.ShapeDtypeStruct((8, n), jnp.float32),
      mesh=mesh,
      compiler_params=pltpu.CompilerParams(needs_layout_passes=False, use_tc_tiling_on_sc=False),
      scratch_types=[
          pltpu.VMEM(xd_flat.shape, jnp.float32),
          pltpu.VMEM(xli.shape, jnp.int32),
          pltpu.VMEM(radii_flat.shape, jnp.float32),
          pltpu.VMEM((_LANES,), jnp.int32),
          pltpu.VMEM((8, _LANES), jnp.float32),
      ],
  )(ndi, xd_flat, xli, radii_flat)


def _k2_body(types_ref, nft_ref, emb16_ref, wf8_ref, bfilm_ref, s8_ref,
             wr2t_ref, wsrc_ref, bsrc_ref, wdst_ref, bdst_ref,
             hl_ref, hr_ref):
  n = types_ref.shape[0]
  f32 = jnp.float32

  tt = types_ref[...]
  onehot = (lax.broadcasted_iota(jnp.int32, (n, 16), 1) == tt).astype(f32)
  emb = jnp.dot(onehot, emb16_ref[...], preferred_element_type=f32)

  nft = nft_ref[...]
  film = lax.dot_general(nft, wf8_ref[...], (((0,), (0,)), ((), ())),
                         preferred_element_type=f32) + bfilm_ref[...]
  gamma = film[:, :128]
  beta = film[:, 128:]
  x_emb = gamma * emb + beta

  c0 = nft[0:1, :]
  c1 = nft[1:2, :]
  r1i = nft[2:3, :]
  r1o = nft[3:4, :]
  r2i = nft[4:5, :]
  r2o = nft[5:6, :]
  s = s8_ref[...]
  sqrt2 = np.sqrt(2.0).astype(np.float32)
  c_eup = np.sqrt(2.0 / np.pi).astype(np.float32)

  def fk(x):
    return x * erf(x / (sqrt2 * s)) + s * c_eup * jnp.exp(-(x * x) / (2.0 * s * s))

  def g(a, b):
    return fk(a + b) - fk(a - b)

  integ = g(r1o, r2o) - g(r1i, r2o) - g(r1o, r2i) + g(r1i, r2i)
  integ = integ * (c0 * c1)
  integ = jnp.maximum(integ, 0.0)
  rowmask = (lax.broadcasted_iota(jnp.int32, (8, 1), 0) < 5).astype(f32)
  integ = integ * rowmask

  inv_n = np.float32(1.0 / n)
  mean = jnp.sum(integ, axis=1, keepdims=True) * inv_n
  var = jnp.sum((integ - mean) * (integ - mean), axis=1, keepdims=True) * inv_n
  scale = 1.0 / (jnp.sqrt(var) + 1e-5)

  wr2t = wr2t_ref[...]
  ones_row = jnp.ones((1, 128), f32)
  na = jnp.zeros((n, 128), f32)
  for i in range(5):
    bi = lax.dot_general(integ[i:i + 1, :] * scale[i:i + 1, :], ones_row,
                         (((0,), (0,)), ((), ())), preferred_element_type=f32)
    t2i = jnp.dot(x_emb, wr2t[:, i * 128:(i + 1) * 128],
                  preferred_element_type=f32)
    na = na + bi * t2i

  hl_ref[...] = jnp.dot(na, wsrc_ref[...], preferred_element_type=f32) + bsrc_ref[...]
  hr_ref[...] = jnp.dot(na, wdst_ref[...], preferred_element_type=f32) + bdst_ref[...]


def _k2_call(types2d, nft, emb16, wf8, bfilm_row, s8, wr2t,
             wsrc, bsrc_row, wdst, bdst_row):
  n = types2d.shape[0]
  d = emb16.shape[1]
  return pl.pallas_call(
      _k2_body,
      out_shape=[jax.ShapeDtypeStruct((n, d), jnp.float32),
                 jax.ShapeDtypeStruct((n, d), jnp.float32)],
      compiler_params=pltpu.CompilerParams(vmem_limit_bytes=120 * 1024 * 1024),
  )(types2d, nft, emb16, wf8, bfilm_row, s8, wr2t, wsrc, bsrc_row, wdst, bdst_row)


_CHUNK = 80


def _k3_body(src_hbm, dst_hbm, hl_hbm, hr_hbm, att_hbm,
             num_out, den_out,
             num_sp, den_sp, zbuf, src_v, dst_v, rows_l, rows_r,
             wbuf16, pbuf, wtmp, att_v):
  cid = lax.axis_index("c")
  sid = lax.axis_index("s")
  wid = sid * _NC + cid

  n = hl_hbm.shape[0]
  e_total = src_hbm.shape[0]
  nw = _NC * _NS16
  e_tile = e_total // nw
  n_chunks = e_tile // _CHUNK
  rows_per_tile = n // _NS16
  f32 = jnp.float32

  zrows = rows_per_tile // 5
  def zfill(i, carry):
    zbuf[i, :] = jnp.zeros((_LANES,), f32)
    return carry
  lax.fori_loop(0, zrows, zfill, 0)
  r0 = sid * rows_per_tile
  for b in range(5):
    rb = r0 + b * zrows
    for c in range(8):
      pltpu.sync_copy(zbuf, num_sp.at[pl.ds(rb, zrows), pl.ds(c * 16, 16)])
    pltpu.sync_copy(zbuf, den_sp.at[pl.ds(rb, zrows), :])

  pltpu.sync_copy(att_hbm, att_v)
  att_c = [att_v[pl.ds(16 * k, 16)] for k in range(8)]
  plsc.subcore_barrier()

  lane_iota16 = lax.iota(jnp.int32, _LANES) * 16

  def chunk_body(ch, carry):
    eb = wid * e_tile + ch * _CHUNK
    pltpu.sync_copy(src_hbm.at[pl.ds(eb, _CHUNK)], src_v)
    pltpu.sync_copy(dst_hbm.at[pl.ds(eb, _CHUNK)], dst_v)
    pltpu.sync_copy(hl_hbm.at[src_v], rows_l)
    pltpu.sync_copy(hr_hbm.at[dst_v], rows_r)

    def group_body(gi, gcarry):
      e0 = gi * _LANES
      for j in range(_LANES):
        p = jnp.zeros((_LANES,), f32)
        for k in range(8):
          t = rows_l[e0 + j, pl.ds(16 * k, 16)] + rows_r[e0 + j, pl.ds(16 * k, 16)]
          m = jnp.maximum(t, 0.2 * t)
          p = p + att_c[k] * m
        pbuf[pl.ds(j * 16, 16)] = p
      sc = plsc.load_gather(pbuf, [lane_iota16])
      for l in range(1, _LANES):
        sc = sc + plsc.load_gather(pbuf, [lane_iota16 + l])
      w16 = jnp.exp(sc)
      wtmp[...] = w16
      for j in range(_LANES):
        wv = plsc.load_gather(wtmp, [jnp.full((_LANES,), j, jnp.int32)])
        wbuf16[e0 + j, :] = wv
        for k in range(8):
          rows_l[e0 + j, pl.ds(16 * k, 16)] = rows_l[e0 + j, pl.ds(16 * k, 16)] * wv
      return gcarry

    lax.fori_loop(0, _CHUNK // _LANES, group_body, 0)

    pltpu.sync_copy(rows_l, num_sp.at[dst_v], add=True)
    pltpu.sync_copy(wbuf16, den_sp.at[dst_v], add=True)
    return carry

  lax.fori_loop(0, n_chunks, chunk_body, 0)
  plsc.subcore_barrier()

  pltpu.sync_copy(num_sp.at[pl.ds(r0, rows_per_tile), :],
                  num_out.at[cid, pl.ds(r0, rows_per_tile), :])
  pltpu.sync_copy(den_sp.at[pl.ds(r0, rows_per_tile), :],
                  den_out.at[cid, pl.ds(r0, rows_per_tile), :])


def _k3_call(src, dst, hl, hr, att):
  n, d = hl.shape
  mesh = plsc.VectorSubcoreMesh(core_axis_name="c", subcore_axis_name="s",
                                num_cores=_NC, num_subcores=_NS16)
  return pl.kernel(
      _k3_body,
      out_type=[jax.ShapeDtypeStruct((_NC, n, d), jnp.float32),
                jax.ShapeDtypeStruct((_NC, n, 16), jnp.float32)],
      mesh=mesh,
      compiler_params=pltpu.CompilerParams(needs_layout_passes=False, use_tc_tiling_on_sc=False),
      scratch_types=[
          pltpu.VMEM_SHARED((n, d), jnp.float32),
          pltpu.VMEM_SHARED((n, 16), jnp.float32),
          pltpu.VMEM((n // _NS16 // 5, 16), jnp.float32),
          pltpu.VMEM((_CHUNK,), jnp.int32),
          pltpu.VMEM((_CHUNK,), jnp.int32),
          pltpu.VMEM((_CHUNK, d), jnp.float32),
          pltpu.VMEM((_CHUNK, d), jnp.float32),
          pltpu.VMEM((_CHUNK, 16), jnp.float32),
          pltpu.VMEM((_LANES * _LANES,), jnp.float32),
          pltpu.VMEM((_LANES,), jnp.float32),
          pltpu.VMEM((d,), jnp.float32),
      ],
  )(src, dst, hl, hr, att)


def _k4_body(num_ref, den_ref, batch_ref, bias_ref, out_ref):
  f32 = jnp.float32
  n = batch_ref.shape[0]
  num = num_ref[0] + num_ref[1]
  den = den_ref[0, :, 0:1] + den_ref[1, :, 0:1]
  msg = num / (den + 1e-16) + bias_ref[...]
  oh = (lax.broadcasted_iota(jnp.int32, (n, _NG), 1) == batch_ref[...]).astype(f32)
  gsum = lax.dot_general(oh, msg, (((0,), (0,)), ((), ())),
                         preferred_element_type=f32)
  nrm = jnp.sqrt(jnp.sum(gsum * gsum, axis=1, keepdims=True))
  out_ref[...] = gsum / jnp.maximum(nrm, 1e-12)


def _k4_call(num2, den2, batch2d, bias_row):
  d = num2.shape[2]
  return pl.pallas_call(
      _k4_body,
      out_shape=jax.ShapeDtypeStruct((_NG, d), jnp.float32),
      compiler_params=pltpu.CompilerParams(vmem_limit_bytes=120 * 1024 * 1024),
  )(num2, den2, batch2d, bias_row)


def kernel(types, x_dopant, node_dopant_index, radii, x_layer_idx, edge_index,
           batch, embed_table, W_film, b_film, sigma, W_bilin, W_src, b_src,
           W_dst, b_dst, att, bias_out):
  n = types.shape[0]
  v, d = embed_table.shape
  ns = sigma.shape[0]
  f32 = jnp.float32

  ndi = node_dopant_index.astype(jnp.int32)
  xd_flat = x_dopant.reshape(-1)
  xli = x_layer_idx.astype(jnp.int32)
  radii_flat = radii.reshape(-1)

  nft = _k1_call(ndi, xd_flat, xli, radii_flat)

  types2d = types.astype(jnp.int32).reshape(n, 1)
  emb16 = jnp.pad(embed_table, ((0, 16 - v), (0, 0)))
  wf8 = jnp.pad(W_film, ((0, 6), (0, 0)))
  bfilm_row = b_film.reshape(1, -1)
  s8 = jnp.concatenate([jnp.maximum(sigma, 1e-4),
                        jnp.ones((8 - ns,), f32)]).reshape(8, 1)
  wr2t = W_bilin.transpose(2, 1, 0).reshape(d, ns * d)
  bsrc_row = b_src.reshape(1, -1)
  bdst_row = b_dst.reshape(1, -1)

  hl, hr = _k2_call(types2d, nft, emb16, wf8, bfilm_row, s8, wr2t,
                    W_src, bsrc_row, W_dst, bdst_row)

  src = edge_index[0].astype(jnp.int32)
  dst = edge_index[1].astype(jnp.int32)
  num2, den2 = _k3_call(src, dst, hl, hr, att)

  batch2d = batch.astype(jnp.int32).reshape(n, 1)
  bias_row = bias_out.reshape(1, -1)
  return _k4_call(num2, den2, batch2d, bias_row)


# speedup vs baseline: 15.5548x; 2.0318x over previous
import functools

import jax
import jax.numpy as jnp
import numpy as np
from jax import lax
from jax.experimental import pallas as pl
from jax.experimental.pallas import tpu as pltpu
from jax.experimental.pallas import tpu_sc as plsc
from jax.scipy.special import erf

_NC = 2
_NS16 = 16
_LANES = 16

_NG = 256


def _k1_body(ndi_hbm, xd_hbm, xli_hbm, radii_hbm, out_hbm,
             xd_v, xli_v, radii_v, idx_v, nfbuf_v):
  cid = lax.axis_index("c")
  sid = lax.axis_index("s")
  wid = sid * _NC + cid

  n = ndi_hbm.shape[0]
  n_groups = n // _LANES
  nw = _NC * _NS16
  base_g = n_groups // nw
  extra = n_groups % nw
  n_g = base_g + jnp.where(wid < extra, 1, 0)

  pltpu.sync_copy(xd_hbm, xd_v)
  pltpu.sync_copy(xli_hbm, xli_v)
  pltpu.sync_copy(radii_hbm, radii_v)

  def group_body(t, carry):
    g = wid + t * nw
    base = g * _LANES
    pltpu.sync_copy(ndi_hbm.at[pl.ds(base, _LANES)], idx_v)
    idx = idx_v[...]
    c0 = plsc.load_gather(xd_v, [idx * 2])
    c1 = plsc.load_gather(xd_v, [idx * 2 + 1])
    li = plsc.load_gather(xli_v, [idx])
    r0 = plsc.load_gather(radii_v, [li * 4])
    r1 = plsc.load_gather(radii_v, [li * 4 + 1])
    r2 = plsc.load_gather(radii_v, [li * 4 + 2])
    r3 = plsc.load_gather(radii_v, [li * 4 + 3])
    nfbuf_v[0, :] = c0
    nfbuf_v[1, :] = c1
    nfbuf_v[2, :] = r0
    nfbuf_v[3, :] = r1
    nfbuf_v[4, :] = r2
    nfbuf_v[5, :] = r3
    z = jnp.zeros((_LANES,), jnp.float32)
    nfbuf_v[6, :] = z
    nfbuf_v[7, :] = z
    pltpu.sync_copy(nfbuf_v, out_hbm.at[:, pl.ds(base, _LANES)])
    return carry

  lax.fori_loop(0, n_g, group_body, 0)


def _k1_call(ndi, xd_flat, xli, radii_flat):
  n = ndi.shape[0]
  mesh = plsc.VectorSubcoreMesh(core_axis_name="c", subcore_axis_name="s",
                                num_cores=_NC, num_subcores=_NS16)
  return pl.kernel(
      _k1_body,
      out_type=jax.ShapeDtypeStruct((8, n), jnp.float32),
      mesh=mesh,
      compiler_params=pltpu.CompilerParams(needs_layout_passes=False, use_tc_tiling_on_sc=False),
      scratch_types=[
          pltpu.VMEM(xd_flat.shape, jnp.float32),
          pltpu.VMEM(xli.shape, jnp.int32),
          pltpu.VMEM(radii_flat.shape, jnp.float32),
          pltpu.VMEM((_LANES,), jnp.int32),
          pltpu.VMEM((8, _LANES), jnp.float32),
      ],
  )(ndi, xd_flat, xli, radii_flat)


def _k2_body(types_ref, nft_ref, emb16_ref, wf8_ref, bfilm_ref, s8_ref,
             wr2t_ref, wsrc_ref, bsrc_ref, wdst_ref, bdst_ref,
             hl_ref, hr_ref):
  n = types_ref.shape[0]
  f32 = jnp.float32

  tt = types_ref[...]
  onehot = (lax.broadcasted_iota(jnp.int32, (n, 16), 1) == tt).astype(f32)
  emb = jnp.dot(onehot, emb16_ref[...], preferred_element_type=f32)

  nft = nft_ref[...]
  film = lax.dot_general(nft, wf8_ref[...], (((0,), (0,)), ((), ())),
                         preferred_element_type=f32) + bfilm_ref[...]
  gamma = film[:, :128]
  beta = film[:, 128:]
  x_emb = gamma * emb + beta

  c0 = nft[0:1, :]
  c1 = nft[1:2, :]
  r1i = nft[2:3, :]
  r1o = nft[3:4, :]
  r2i = nft[4:5, :]
  r2o = nft[5:6, :]
  s = s8_ref[...]
  sqrt2 = np.sqrt(2.0).astype(np.float32)
  c_eup = np.sqrt(2.0 / np.pi).astype(np.float32)

  def fk(x):
    return x * erf(x / (sqrt2 * s)) + s * c_eup * jnp.exp(-(x * x) / (2.0 * s * s))

  def g(a, b):
    return fk(a + b) - fk(a - b)

  integ = g(r1o, r2o) - g(r1i, r2o) - g(r1o, r2i) + g(r1i, r2i)
  integ = integ * (c0 * c1)
  integ = jnp.maximum(integ, 0.0)
  rowmask = (lax.broadcasted_iota(jnp.int32, (8, 1), 0) < 5).astype(f32)
  integ = integ * rowmask

  inv_n = np.float32(1.0 / n)
  mean = jnp.sum(integ, axis=1, keepdims=True) * inv_n
  var = jnp.sum((integ - mean) * (integ - mean), axis=1, keepdims=True) * inv_n
  scale = 1.0 / (jnp.sqrt(var) + 1e-5)

  wr2t = wr2t_ref[...]
  ones_row = jnp.ones((1, 128), f32)
  na = jnp.zeros((n, 128), f32)
  for i in range(5):
    bi = lax.dot_general(integ[i:i + 1, :] * scale[i:i + 1, :], ones_row,
                         (((0,), (0,)), ((), ())), preferred_element_type=f32)
    t2i = jnp.dot(x_emb, wr2t[:, i * 128:(i + 1) * 128],
                  preferred_element_type=f32)
    na = na + bi * t2i

  hl_ref[...] = jnp.dot(na, wsrc_ref[...], preferred_element_type=f32) + bsrc_ref[...]
  hr_ref[...] = jnp.dot(na, wdst_ref[...], preferred_element_type=f32) + bdst_ref[...]


def _k2_call(types2d, nft, emb16, wf8, bfilm_row, s8, wr2t,
             wsrc, bsrc_row, wdst, bdst_row):
  n = types2d.shape[0]
  d = emb16.shape[1]
  return pl.pallas_call(
      _k2_body,
      out_shape=[jax.ShapeDtypeStruct((n, d), jnp.float32),
                 jax.ShapeDtypeStruct((n, d), jnp.float32)],
      compiler_params=pltpu.CompilerParams(vmem_limit_bytes=120 * 1024 * 1024),
  )(types2d, nft, emb16, wf8, bfilm_row, s8, wr2t, wsrc, bsrc_row, wdst, bdst_row)


_CHUNK = 48
_DEPTH = 3
_IDEPTH = 6


def _k3_body(ei_hbm, hl_hbm, hr_hbm, att_hbm,
             num_out, den_out,
             num_sp, zbuf, den_v, pbuf, wbuf, wtmp, att_v, tidx,
             ib0, ib1, ib2, ib3, ib4, ib5, rl0, rl1, rl2, rr0, rr1, rr2,
             si0, si1, si2, si3, si4, si5, sgl0, sgl1, sgl2,
             sgr0, sgr1, sgr2, ss0, ss1, ss2):
  cid = lax.axis_index("c")
  sid = lax.axis_index("s")
  wid = sid * _NC + cid

  n = hl_hbm.shape[0]
  e_total = ei_hbm.shape[1]
  nw = _NC * _NS16
  e_tile = e_total // nw
  n_chunks = e_tile // _CHUNK
  tail = e_tile - n_chunks * _CHUNK
  rows_per_tile = n // _NS16
  f32 = jnp.float32
  ebase = wid * e_tile

  idxb = [ib0, ib1, ib2, ib3, ib4, ib5]
  rl = [rl0, rl1, rl2]
  rr = [rr0, rr1, rr2]
  sem_i = [si0, si1, si2, si3, si4, si5]
  sem_gl = [sgl0, sgl1, sgl2]
  sem_gr = [sgr0, sgr1, sgr2]
  sem_s = [ss0, ss1, ss2]

  zrows = rows_per_tile // 5
  def zfill(i, carry):
    zbuf[i, :] = jnp.zeros((_LANES,), f32)
    return carry
  lax.fori_loop(0, zrows, zfill, 0)
  r0 = sid * rows_per_tile
  for b in range(5):
    rb = r0 + b * zrows
    for c in range(8):
      pltpu.sync_copy(zbuf, num_sp.at[pl.ds(rb, zrows), pl.ds(c * 16, 16)])
  def dzfill(i, carry):
    den_v[pl.ds(i * _LANES, _LANES)] = jnp.zeros((_LANES,), f32)
    return carry
  lax.fori_loop(0, n // _LANES, dzfill, 0)

  pltpu.sync_copy(att_hbm, att_v)
  att_c = [att_v[pl.ds(16 * k, 16)] for k in range(8)]
  plsc.subcore_barrier()

  lane_iota16 = lax.iota(jnp.int32, _LANES) * 16

  def idx_start(ch, slot):
    pltpu.async_copy(ei_hbm.at[:, pl.ds(ebase + ch * _CHUNK, _CHUNK)],
                     idxb[slot], sem_i[slot])

  def idx_wait(ch, slot):
    pltpu.make_async_copy(ei_hbm.at[:, pl.ds(ebase + ch * _CHUNK, _CHUNK)],
                          idxb[slot], sem_i[slot]).wait()

  def gathers_start(b, ib):
    pltpu.async_copy(hl_hbm.at[idxb[ib].at[0]], rl[b], sem_gl[b])
    pltpu.async_copy(hr_hbm.at[idxb[ib].at[1]], rr[b], sem_gr[b])

  def gathers_wait(b, ib):
    pltpu.make_async_copy(hl_hbm.at[idxb[ib].at[0]], rl[b], sem_gl[b]).wait()
    pltpu.make_async_copy(hr_hbm.at[idxb[ib].at[1]], rr[b], sem_gr[b]).wait()

  def scatter_start(b, ib):
    pltpu.async_copy(rl[b], num_sp.at[idxb[ib].at[1]], sem_s[b], add=True)

  def scatter_wait(b, ib):
    pltpu.make_async_copy(rl[b], num_sp.at[idxb[ib].at[1]], sem_s[b]).wait()

  def compute(rows_l, rows_r, idxref, n_edges):
    def phase_a(i, carry):
      for jj in range(4):
        j = i * 4 + jj
        p = jnp.zeros((_LANES,), f32)
        for k in range(8):
          t = rows_l[j, pl.ds(16 * k, 16)] + rows_r[j, pl.ds(16 * k, 16)]
          m = jnp.maximum(t, 0.2 * t)
          p = p + att_c[k] * m
        pbuf[pl.ds(j * 16, 16)] = p
      return carry
    lax.fori_loop(0, n_edges // 4, phase_a, 0)
    for gi in range(n_edges // _LANES):
      base = gi * 256
      sc = plsc.load_gather(pbuf, [lane_iota16 + base])
      for l in range(1, _LANES):
        sc = sc + plsc.load_gather(pbuf, [lane_iota16 + base + l])
      w16 = jnp.exp(sc)
      dst16 = idxref[1, pl.ds(gi * _LANES, _LANES)]
      plsc.addupdate_scatter(den_v, [dst16], w16)
      wbuf[pl.ds(gi * _LANES, _LANES)] = w16
    def phase_c(i, carry):
      for jj in range(2):
        j = i * 2 + jj
        wv = plsc.load_gather(wbuf, [jnp.zeros((_LANES,), jnp.int32) + j])
        for k in range(8):
          rows_l[j, pl.ds(16 * k, 16)] = rows_l[j, pl.ds(16 * k, 16)] * wv
      return carry
    lax.fori_loop(0, n_edges // 2, phase_c, 0)

  n_super = (n_chunks + _IDEPTH - 1) // _IDEPTH
  idx_start(0, 0)
  if n_chunks > 1:
    idx_start(1, 1)
  idx_wait(0, 0)
  gathers_start(0, 0)

  def superstep(s, carry):
    c0 = s * _IDEPTH
    for j in range(_IDEPTH):
      c = c0 + j
      b = j % _DEPTH
      ib = j
      b1 = (j + 1) % _DEPTH
      ib1 = (j + 1) % _IDEPTH
      b2 = (j + 1) % _DEPTH
      ib2 = (j + 4) % _IDEPTH
      ib2f = (j + 2) % _IDEPTH

      @pl.when(c < n_chunks)
      def _(c=c, b=b, ib=ib, b1=b1, ib1=ib1, b2=b2, ib2=ib2, ib2f=ib2f):
        gathers_wait(b, ib)
        @pl.when(c >= 2)
        def _():
          scatter_wait(b2, ib2)
        @pl.when(c + 1 < n_chunks)
        def _():
          idx_wait(c + 1, ib1)
          gathers_start(b1, ib1)
        @pl.when(c + 2 < n_chunks)
        def _():
          idx_start(c + 2, ib2f)
        compute(rl[b], rr[b], idxb[ib], _CHUNK)
        scatter_start(b, ib)
    return carry

  lax.fori_loop(0, n_super, superstep, 0)

  for last in (n_chunks - 2, n_chunks - 1):
    if last >= 0:
      j = last % _IDEPTH
      scatter_wait(j % _DEPTH, j)

  if tail > 0:
    tbase = ebase + n_chunks * _CHUNK
    pltpu.sync_copy(ei_hbm.at[:, pl.ds(tbase, tail)], tidx)
    pltpu.sync_copy(hl_hbm.at[tidx.at[0]], rl[0].at[pl.ds(0, tail), :])
    pltpu.sync_copy(hr_hbm.at[tidx.at[1]], rr[0].at[pl.ds(0, tail), :])
    compute(rl[0], rr[0], tidx, tail)
    pltpu.sync_copy(rl[0].at[pl.ds(0, tail), :], num_sp.at[tidx.at[1]],
                    add=True)

  plsc.subcore_barrier()

  pltpu.sync_copy(num_sp.at[pl.ds(r0, rows_per_tile), :],
                  num_out.at[cid, pl.ds(r0, rows_per_tile), :])
  pltpu.sync_copy(den_v, den_out.at[cid, sid, :])


def _k3_call(edge_index, hl, hr, att):
  n, d = hl.shape
  e_total = edge_index.shape[1]
  tail = (e_total // (_NC * _NS16)) % _CHUNK
  mesh = plsc.VectorSubcoreMesh(core_axis_name="c", subcore_axis_name="s",
                                num_cores=_NC, num_subcores=_NS16)
  idx_bufs = [pltpu.VMEM((2, _CHUNK), jnp.int32) for _ in range(_IDEPTH)]
  row_bufs = [pltpu.VMEM((_CHUNK, d), jnp.float32) for _ in range(2 * _DEPTH)]
  sems = [pltpu.SemaphoreType.DMA for _ in range(_IDEPTH + 3 * _DEPTH)]
  return pl.kernel(
      _k3_body,
      out_type=[jax.ShapeDtypeStruct((_NC, n, d), jnp.float32),
                jax.ShapeDtypeStruct((_NC, _NS16, n), jnp.float32)],
      mesh=mesh,
      compiler_params=pltpu.CompilerParams(needs_layout_passes=False, use_tc_tiling_on_sc=False),
      scratch_types=[
          pltpu.VMEM_SHARED((n, d), jnp.float32),
          pltpu.VMEM((n // _NS16 // 5, 16), jnp.float32),
          pltpu.VMEM((n,), jnp.float32),
          pltpu.VMEM((_CHUNK * _LANES,), jnp.float32),
          pltpu.VMEM((_CHUNK,), jnp.float32),
          pltpu.VMEM((_LANES,), jnp.float32),
          pltpu.VMEM((d,), jnp.float32),
          pltpu.VMEM((2, max(tail, _LANES)), jnp.int32),
      ] + idx_bufs + row_bufs + sems,
  )(edge_index, hl, hr, att)


def _k4_body(num_ref, den_ref, batch_ref, bias_ref, out_ref):
  f32 = jnp.float32
  n = batch_ref.shape[0]
  num = num_ref[0] + num_ref[1]
  dall = den_ref[...].reshape(2 * _NS16, n)
  drow = jnp.sum(dall, axis=0, keepdims=True)
  den = lax.dot_general(drow, jnp.ones((1, 1), f32),
                        (((0,), (0,)), ((), ())),
                        preferred_element_type=f32)
  msg = num / (den + 1e-16) + bias_ref[...]
  oh = (lax.broadcasted_iota(jnp.int32, (n, _NG), 1) == batch_ref[...]).astype(f32)
  gsum = lax.dot_general(oh, msg, (((0,), (0,)), ((), ())),
                         preferred_element_type=f32)
  nrm = jnp.sqrt(jnp.sum(gsum * gsum, axis=1, keepdims=True))
  out_ref[...] = gsum / jnp.maximum(nrm, 1e-12)


def _k4_call(num2, den2, batch2d, bias_row):
  d = num2.shape[2]
  return pl.pallas_call(
      _k4_body,
      out_shape=jax.ShapeDtypeStruct((_NG, d), jnp.float32),
      compiler_params=pltpu.CompilerParams(vmem_limit_bytes=120 * 1024 * 1024),
  )(num2, den2, batch2d, bias_row)


def kernel(types, x_dopant, node_dopant_index, radii, x_layer_idx, edge_index,
           batch, embed_table, W_film, b_film, sigma, W_bilin, W_src, b_src,
           W_dst, b_dst, att, bias_out):
  n = types.shape[0]
  v, d = embed_table.shape
  ns = sigma.shape[0]
  f32 = jnp.float32

  ndi = node_dopant_index.astype(jnp.int32)
  xd_flat = x_dopant.reshape(-1)
  xli = x_layer_idx.astype(jnp.int32)
  radii_flat = radii.reshape(-1)

  nft = _k1_call(ndi, xd_flat, xli, radii_flat)

  types2d = types.astype(jnp.int32).reshape(n, 1)
  emb16 = jnp.pad(embed_table, ((0, 16 - v), (0, 0)))
  wf8 = jnp.pad(W_film, ((0, 6), (0, 0)))
  bfilm_row = b_film.reshape(1, -1)
  s8 = jnp.concatenate([jnp.maximum(sigma, 1e-4),
                        jnp.ones((8 - ns,), f32)]).reshape(8, 1)
  wr2t = W_bilin.transpose(2, 1, 0).reshape(d, ns * d)
  bsrc_row = b_src.reshape(1, -1)
  bdst_row = b_dst.reshape(1, -1)

  hl, hr = _k2_call(types2d, nft, emb16, wf8, bfilm_row, s8, wr2t,
                    W_src, bsrc_row, W_dst, bdst_row)

  num2, den2 = _k3_call(edge_index.astype(jnp.int32), hl, hr, att)

  batch2d = batch.astype(jnp.int32).reshape(n, 1)
  bias_row = bias_out.reshape(1, -1)
  return _k4_call(num2, den2, batch2d, bias_row)


# speedup vs baseline: 19.1310x; 1.2299x over previous
import functools

import jax
import jax.numpy as jnp
import numpy as np
from jax import lax
from jax.experimental import pallas as pl
from jax.experimental.pallas import tpu as pltpu
from jax.experimental.pallas import tpu_sc as plsc
from jax.scipy.special import erf

_NC = 2
_NS16 = 16
_LANES = 16

_NG = 256


def _k1_body(ndi_hbm, xd_hbm, xli_hbm, radii_hbm, out_hbm,
             xd_v, xli_v, radii_v, ndi_v, nfbuf_v):
  cid = lax.axis_index("c")
  sid = lax.axis_index("s")
  wid = sid * _NC + cid

  n = ndi_hbm.shape[0]
  nw = _NC * _NS16
  n_tile = n // nw
  base = wid * n_tile

  pltpu.sync_copy(xd_hbm, xd_v)
  pltpu.sync_copy(xli_hbm, xli_v)
  pltpu.sync_copy(radii_hbm, radii_v)
  pltpu.sync_copy(ndi_hbm.at[pl.ds(base, n_tile)], ndi_v)

  z = jnp.zeros((_LANES,), jnp.float32)

  def group_body(g):
    idx = ndi_v[pl.ds(g * _LANES, _LANES)]
    c0 = plsc.load_gather(xd_v, [idx * 2])
    c1 = plsc.load_gather(xd_v, [idx * 2 + 1])
    li = plsc.load_gather(xli_v, [idx])
    r0 = plsc.load_gather(radii_v, [li * 4])
    r1 = plsc.load_gather(radii_v, [li * 4 + 1])
    r2 = plsc.load_gather(radii_v, [li * 4 + 2])
    r3 = plsc.load_gather(radii_v, [li * 4 + 3])
    sl = pl.ds(g * _LANES, _LANES)
    nfbuf_v[0, sl] = c0
    nfbuf_v[1, sl] = c1
    nfbuf_v[2, sl] = r0
    nfbuf_v[3, sl] = r1
    nfbuf_v[4, sl] = r2
    nfbuf_v[5, sl] = r3
    nfbuf_v[6, sl] = z
    nfbuf_v[7, sl] = z

  plsc.parallel_loop(0, n_tile // _LANES, 1, unroll=2)(group_body)
  pltpu.sync_copy(nfbuf_v, out_hbm.at[:, pl.ds(base, n_tile)])


def _k1_call(ndi, xd_flat, xli, radii_flat):
  n = ndi.shape[0]
  nw = _NC * _NS16
  mesh = plsc.VectorSubcoreMesh(core_axis_name="c", subcore_axis_name="s",
                                num_cores=_NC, num_subcores=_NS16)
  return pl.kernel(
      _k1_body,
      out_type=jax.ShapeDtypeStruct((8, n), jnp.float32),
      mesh=mesh,
      compiler_params=pltpu.CompilerParams(needs_layout_passes=False, use_tc_tiling_on_sc=False),
      scratch_types=[
          pltpu.VMEM(xd_flat.shape, jnp.float32),
          pltpu.VMEM(xli.shape, jnp.int32),
          pltpu.VMEM(radii_flat.shape, jnp.float32),
          pltpu.VMEM((n // nw,), jnp.int32),
          pltpu.VMEM((8, n // nw), jnp.float32),
      ],
  )(ndi, xd_flat, xli, radii_flat)


def _k2_body(n_valid, types_ref, nft_ref, emb16_ref, wf8_ref, bfilm_ref, s8_ref,
             wr2t_ref, wsrc_ref, bsrc_ref, wdst_ref, bdst_ref,
             hl_ref, hr_ref):
  n = types_ref.shape[0]
  f32 = jnp.float32

  tt = types_ref[...]
  onehot = (lax.broadcasted_iota(jnp.int32, (n, 16), 1) == tt).astype(f32)
  emb = jnp.dot(onehot, emb16_ref[...], preferred_element_type=f32)

  nft = nft_ref[...]
  film = lax.dot_general(nft, wf8_ref[...], (((0,), (0,)), ((), ())),
                         preferred_element_type=f32) + bfilm_ref[...]
  gamma = film[:, :128]
  beta = film[:, 128:]
  x_emb = gamma * emb + beta

  c0 = nft[0:1, :]
  c1 = nft[1:2, :]
  r1i = nft[2:3, :]
  r1o = nft[3:4, :]
  r2i = nft[4:5, :]
  r2o = nft[5:6, :]
  s = s8_ref[...]
  sqrt2 = np.sqrt(2.0).astype(np.float32)
  c_eup = np.sqrt(2.0 / np.pi).astype(np.float32)

  def fk(x):
    return x * erf(x / (sqrt2 * s)) + s * c_eup * jnp.exp(-(x * x) / (2.0 * s * s))

  def g(a, b):
    return fk(a + b) - fk(a - b)

  integ = g(r1o, r2o) - g(r1i, r2o) - g(r1o, r2i) + g(r1i, r2i)
  integ = integ * (c0 * c1)
  integ = jnp.maximum(integ, 0.0)
  rowmask = (lax.broadcasted_iota(jnp.int32, (8, 1), 0) < 5).astype(f32)
  integ = integ * rowmask
  if n_valid < n:
    colmask = (lax.broadcasted_iota(jnp.int32, (8, n), 1) < n_valid).astype(f32)
    integ = integ * colmask

  inv_n = np.float32(1.0 / n_valid)
  mean = jnp.sum(integ, axis=1, keepdims=True) * inv_n
  var = jnp.sum((integ - mean) * (integ - mean), axis=1, keepdims=True) * inv_n
  scale = 1.0 / (jnp.sqrt(var) + 1e-5)

  wr2t = wr2t_ref[...]
  ones_row = jnp.ones((1, 128), f32)
  na = jnp.zeros((n, 128), f32)
  for i in range(5):
    bi = lax.dot_general(integ[i:i + 1, :] * scale[i:i + 1, :], ones_row,
                         (((0,), (0,)), ((), ())), preferred_element_type=f32)
    t2i = jnp.dot(x_emb, wr2t[:, i * 128:(i + 1) * 128],
                  preferred_element_type=f32)
    na = na + bi * t2i

  hl_ref[...] = jnp.dot(na, wsrc_ref[...], preferred_element_type=f32) + bsrc_ref[...]
  hr_ref[...] = jnp.dot(na, wdst_ref[...], preferred_element_type=f32) + bdst_ref[...]


def _k2_call(n_valid, types2d, nft, emb16, wf8, bfilm_row, s8, wr2t,
             wsrc, bsrc_row, wdst, bdst_row):
  n = types2d.shape[0]
  d = emb16.shape[1]
  return pl.pallas_call(
      functools.partial(_k2_body, n_valid),
      out_shape=[jax.ShapeDtypeStruct((n, d), jnp.float32),
                 jax.ShapeDtypeStruct((n, d), jnp.float32)],
      compiler_params=pltpu.CompilerParams(vmem_limit_bytes=120 * 1024 * 1024),
  )(types2d, nft, emb16, wf8, bfilm_row, s8, wr2t, wsrc, bsrc_row, wdst, bdst_row)


_CHUNK = 48
_DEPTH = 3
_IDEPTH = 6


def _k3_body(ei_hbm, hl_hbm, hr_hbm, att_hbm,
             num_out, den_out,
             num_sp, zbuf, den_v, pbuf, wbuf, wtmp, att_v, tidx,
             ib0, ib1, ib2, ib3, ib4, ib5, rl0, rl1, rl2, rr0, rr1, rr2,
             si0, si1, si2, si3, si4, si5, sgl0, sgl1, sgl2,
             sgr0, sgr1, sgr2, ss0, ss1, ss2, sz0):
  cid = lax.axis_index("c")
  sid = lax.axis_index("s")
  wid = sid * _NC + cid

  n = num_sp.shape[0]
  e_total = ei_hbm.shape[1]
  nw = _NC * _NS16
  e_tile = e_total // nw
  n_chunks = e_tile // _CHUNK
  tail = e_tile - n_chunks * _CHUNK
  rows_per_tile = n // _NS16
  f32 = jnp.float32
  ebase = wid * e_tile

  idxb = [ib0, ib1, ib2, ib3, ib4, ib5]
  rl = [rl0, rl1, rl2]
  rr = [rr0, rr1, rr2]
  sem_i = [si0, si1, si2, si3, si4, si5]
  sem_gl = [sgl0, sgl1, sgl2]
  sem_gr = [sgr0, sgr1, sgr2]
  sem_s = [ss0, ss1, ss2]

  zrows = rows_per_tile // 5
  def zfill(i, carry):
    zbuf[i, :] = jnp.zeros((_LANES,), f32)
    return carry
  lax.fori_loop(0, zrows, zfill, 0)
  r0 = sid * rows_per_tile
  for b in range(5):
    rb = r0 + b * zrows
    for c in range(8):
      pltpu.async_copy(zbuf, num_sp.at[pl.ds(rb, zrows), pl.ds(c * 16, 16)], sz0)
    for c in range(8):
      pltpu.make_async_copy(
          zbuf, num_sp.at[pl.ds(rb, zrows), pl.ds(c * 16, 16)], sz0).wait()
  def dzfill(i, carry):
    den_v[pl.ds(i * _LANES, _LANES)] = jnp.zeros((_LANES,), f32)
    return carry
  lax.fori_loop(0, n // _LANES, dzfill, 0)

  pltpu.sync_copy(att_hbm, att_v)
  att_c = [att_v[pl.ds(16 * k, 16)] for k in range(8)]
  plsc.subcore_barrier()

  lane_iota16 = lax.iota(jnp.int32, _LANES) * 16

  def idx_start(ch, slot):
    pltpu.async_copy(ei_hbm.at[:, pl.ds(ebase + ch * _CHUNK, _CHUNK)],
                     idxb[slot], sem_i[slot])

  def idx_wait(ch, slot):
    pltpu.make_async_copy(ei_hbm.at[:, pl.ds(ebase + ch * _CHUNK, _CHUNK)],
                          idxb[slot], sem_i[slot]).wait()

  def gathers_start(b, ib):
    pltpu.async_copy(hl_hbm.at[idxb[ib].at[0]], rl[b], sem_gl[b])
    pltpu.async_copy(hr_hbm.at[idxb[ib].at[1]], rr[b], sem_gr[b])

  def gathers_wait(b, ib):
    pltpu.make_async_copy(hl_hbm.at[idxb[ib].at[0]], rl[b], sem_gl[b]).wait()
    pltpu.make_async_copy(hr_hbm.at[idxb[ib].at[1]], rr[b], sem_gr[b]).wait()

  def scatter_start(b, ib):
    pltpu.async_copy(rl[b], num_sp.at[idxb[ib].at[1]], sem_s[b], add=True)

  def scatter_wait(b, ib):
    pltpu.make_async_copy(rl[b], num_sp.at[idxb[ib].at[1]], sem_s[b]).wait()

  def compute(rows_l, rows_r, idxref, n_edges):
    def phase_a(j):
      p = jnp.zeros((_LANES,), f32)
      for k in range(8):
        t = rows_l[j, pl.ds(16 * k, 16)] + rows_r[j, pl.ds(16 * k, 16)]
        m = jnp.maximum(t, 0.2 * t)
        p = p + att_c[k] * m
      pbuf[pl.ds(j * 16, 16)] = p
    plsc.parallel_loop(0, n_edges, 1, unroll=4)(phase_a)
    for gi in range(n_edges // _LANES):
      base = gi * 256
      sc = plsc.load_gather(pbuf, [lane_iota16 + base])
      for l in range(1, _LANES):
        sc = sc + plsc.load_gather(pbuf, [lane_iota16 + base + l])
      w16 = jnp.exp(sc)
      dst16 = idxref[1, pl.ds(gi * _LANES, _LANES)]
      plsc.addupdate_scatter(den_v, [dst16], w16)
      wbuf[pl.ds(gi * _LANES, _LANES)] = w16
    def phase_c(j):
      wv = plsc.load_gather(wbuf, [jnp.zeros((_LANES,), jnp.int32) + j])
      for k in range(8):
        rows_l[j, pl.ds(16 * k, 16)] = rows_l[j, pl.ds(16 * k, 16)] * wv
    plsc.parallel_loop(0, n_edges, 1, unroll=4)(phase_c)

  n_super = (n_chunks + _IDEPTH - 1) // _IDEPTH
  idx_start(0, 0)
  if n_chunks > 1:
    idx_start(1, 1)
  idx_wait(0, 0)
  gathers_start(0, 0)

  def superstep(s, carry):
    c0 = s * _IDEPTH
    for j in range(_IDEPTH):
      c = c0 + j
      b = j % _DEPTH
      ib = j
      b1 = (j + 1) % _DEPTH
      ib1 = (j + 1) % _IDEPTH
      b2 = (j + 1) % _DEPTH
      ib2 = (j + 4) % _IDEPTH
      ib2f = (j + 2) % _IDEPTH

      @pl.when(c < n_chunks)
      def _(c=c, b=b, ib=ib, b1=b1, ib1=ib1, b2=b2, ib2=ib2, ib2f=ib2f):
        gathers_wait(b, ib)
        @pl.when(c >= 2)
        def _():
          scatter_wait(b2, ib2)
        @pl.when(c + 1 < n_chunks)
        def _():
          idx_wait(c + 1, ib1)
          gathers_start(b1, ib1)
        @pl.when(c + 2 < n_chunks)
        def _():
          idx_start(c + 2, ib2f)
        compute(rl[b], rr[b], idxb[ib], _CHUNK)
        scatter_start(b, ib)
    return carry

  lax.fori_loop(0, n_super, superstep, 0)

  for last in (n_chunks - 2, n_chunks - 1):
    if last >= 0:
      j = last % _IDEPTH
      scatter_wait(j % _DEPTH, j)

  if tail > 0:
    tbase = ebase + n_chunks * _CHUNK
    pltpu.sync_copy(ei_hbm.at[:, pl.ds(tbase, tail)], tidx)
    pltpu.sync_copy(hl_hbm.at[tidx.at[0]], rl[0].at[pl.ds(0, tail), :])
    pltpu.sync_copy(hr_hbm.at[tidx.at[1]], rr[0].at[pl.ds(0, tail), :])
    compute(rl[0], rr[0], tidx, tail)
    pltpu.sync_copy(rl[0].at[pl.ds(0, tail), :], num_sp.at[tidx.at[1]],
                    add=True)

  plsc.subcore_barrier()

  pltpu.sync_copy(num_sp.at[pl.ds(r0, rows_per_tile), :],
                  num_out.at[cid, pl.ds(r0, rows_per_tile), :])
  pltpu.sync_copy(den_v, den_out.at[cid, sid, :])


def _k3_call(edge_index, hl, hr, att, n):
  d = hl.shape[1]
  e_total = edge_index.shape[1]
  tail = (e_total // (_NC * _NS16)) % _CHUNK
  mesh = plsc.VectorSubcoreMesh(core_axis_name="c", subcore_axis_name="s",
                                num_cores=_NC, num_subcores=_NS16)
  idx_bufs = [pltpu.VMEM((2, _CHUNK), jnp.int32) for _ in range(_IDEPTH)]
  row_bufs = [pltpu.VMEM((_CHUNK, d), jnp.float32) for _ in range(2 * _DEPTH)]
  sems = [pltpu.SemaphoreType.DMA for _ in range(_IDEPTH + 3 * _DEPTH + 1)]
  return pl.kernel(
      _k3_body,
      out_type=[jax.ShapeDtypeStruct((_NC, n, d), jnp.float32),
                jax.ShapeDtypeStruct((_NC, _NS16, n), jnp.float32)],
      mesh=mesh,
      compiler_params=pltpu.CompilerParams(needs_layout_passes=False, use_tc_tiling_on_sc=False),
      scratch_types=[
          pltpu.VMEM_SHARED((n, d), jnp.float32),
          pltpu.VMEM((n // _NS16 // 5, 16), jnp.float32),
          pltpu.VMEM((n,), jnp.float32),
          pltpu.VMEM((_CHUNK * _LANES,), jnp.float32),
          pltpu.VMEM((_CHUNK,), jnp.float32),
          pltpu.VMEM((_LANES,), jnp.float32),
          pltpu.VMEM((d,), jnp.float32),
          pltpu.VMEM((2, max(tail, _LANES)), jnp.int32),
      ] + idx_bufs + row_bufs + sems,
  )(edge_index, hl, hr, att)


def _k4_body(num_ref, den_ref, batch_ref, bias_ref, out_ref):
  f32 = jnp.float32
  n = batch_ref.shape[0]
  num = num_ref[0] + num_ref[1]
  dall = den_ref[...].reshape(2 * _NS16, n)
  drow = jnp.sum(dall, axis=0, keepdims=True)
  den = lax.dot_general(drow, jnp.ones((1, 1), f32),
                        (((0,), (0,)), ((), ())),
                        preferred_element_type=f32)
  msg = num / (den + 1e-16) + bias_ref[...]
  oh = (lax.broadcasted_iota(jnp.int32, (n, _NG), 1) == batch_ref[...]).astype(f32)
  gsum = lax.dot_general(oh, msg, (((0,), (0,)), ((), ())),
                         preferred_element_type=f32)
  nrm = jnp.sqrt(jnp.sum(gsum * gsum, axis=1, keepdims=True))
  out_ref[...] = gsum / jnp.maximum(nrm, 1e-12)


def _k4_call(num2, den2, batch2d, bias_row):
  d = num2.shape[2]
  return pl.pallas_call(
      _k4_body,
      out_shape=jax.ShapeDtypeStruct((_NG, d), jnp.float32),
      compiler_params=pltpu.CompilerParams(vmem_limit_bytes=120 * 1024 * 1024),
  )(num2, den2, batch2d, bias_row)


def kernel(types, x_dopant, node_dopant_index, radii, x_layer_idx, edge_index,
           batch, embed_table, W_film, b_film, sigma, W_bilin, W_src, b_src,
           W_dst, b_dst, att, bias_out):
  n = types.shape[0]
  v, d = embed_table.shape
  ns = sigma.shape[0]
  f32 = jnp.float32

  ndi = node_dopant_index.astype(jnp.int32)
  xd_flat = x_dopant.reshape(-1)
  xli = x_layer_idx.astype(jnp.int32)
  radii_flat = radii.reshape(-1)

  npad = ((n + 511) // 512) * 512
  ndi_pad = jnp.pad(ndi, (0, npad - n))
  nft = _k1_call(ndi_pad, xd_flat, xli, radii_flat)

  types2d = jnp.pad(types.astype(jnp.int32), (0, npad - n)).reshape(npad, 1)
  emb16 = jnp.pad(embed_table, ((0, 16 - v), (0, 0)))
  wf8 = jnp.pad(W_film, ((0, 6), (0, 0)))
  bfilm_row = b_film.reshape(1, -1)
  s8 = jnp.concatenate([jnp.maximum(sigma, 1e-4),
                        jnp.ones((8 - ns,), f32)]).reshape(8, 1)
  wr2t = W_bilin.transpose(2, 1, 0).reshape(d, ns * d)
  bsrc_row = b_src.reshape(1, -1)
  bdst_row = b_dst.reshape(1, -1)

  hl, hr = _k2_call(n, types2d, nft, emb16, wf8, bfilm_row, s8, wr2t,
                    W_src, bsrc_row, W_dst, bdst_row)

  num2, den2 = _k3_call(edge_index.astype(jnp.int32), hl, hr, att, n)

  batch2d = batch.astype(jnp.int32).reshape(n, 1)
  bias_row = bias_out.reshape(1, -1)
  return _k4_call(num2, den2, batch2d, bias_row)
